# baseline (speedup 1.0000x reference)
import functools

import jax
import jax.numpy as jnp
from jax import lax
from jax.experimental import pallas as pl
from jax.experimental.pallas import tpu as pltpu
from jax.experimental.pallas import tpu_sc as plsc

T = 2048
D = 1024
E = 8
DFF = 2048
S = 2 * T
B = 128
NB = S // B
NT = NB + E - 1
CH = 512


def _route_kernel(g_ref, dest_ref, w_ref, cnt_ref):
    g = g_ref[...].astype(jnp.float32)
    m = jnp.max(g, axis=-1, keepdims=True)
    p = jnp.exp(g - m)
    scores = p / jnp.sum(p, axis=-1, keepdims=True)
    i1 = jnp.argmax(scores, axis=-1)[:, None]
    lanes_t = jax.lax.broadcasted_iota(jnp.int32, (T, E), 1)
    one1 = lanes_t == i1
    w1 = jnp.max(scores, axis=-1, keepdims=True)
    masked = jnp.where(one1, -jnp.inf, scores)
    i2 = jnp.argmax(masked, axis=-1)[:, None]
    w2 = jnp.max(masked, axis=-1, keepdims=True)
    denom = w1 + w2
    w1n, w2n = w1 / denom, w2 / denom

    e_flat = jnp.concatenate([i1, i2], axis=0)
    w_flat = jnp.concatenate([w1n, w2n], axis=0)

    lanes = jax.lax.broadcasted_iota(jnp.int32, (S, E), 1)
    onehot = (lanes == e_flat).astype(jnp.float32)

    r = jax.lax.broadcasted_iota(jnp.int32, (CH, CH), 0)
    c = jax.lax.broadcasted_iota(jnp.int32, (CH, CH), 1)
    tril = (r >= c).astype(jnp.float32)

    carry = jnp.zeros((1, E), jnp.float32)
    blocks = []
    for ci in range(S // CH):
        blk = jax.lax.slice(onehot, (ci * CH, 0), ((ci + 1) * CH, E))
        incl = jax.lax.dot_general(
            tril, blk, (((1,), (0,)), ((), ())),
            preferred_element_type=jnp.float32) + carry
        carry = jax.lax.slice(incl, (CH - 1, 0), (CH, E))
        blocks.append(incl)
    incl_all = jnp.concatenate(blocks, axis=0)

    counts = jnp.broadcast_to(carry, (S, E))
    before = jnp.sum(jnp.where(lanes < e_flat, counts, 0.0), axis=1,
                     keepdims=True)
    own = jnp.sum(jnp.where(lanes == e_flat, incl_all, 0.0), axis=1,
                  keepdims=True)
    dest = (before + own - 1.0).astype(jnp.int32)

    dest_ref[...] = jnp.broadcast_to(dest, (S, E))
    w_ref[...] = jnp.broadcast_to(w_flat, (S, E))
    cnt_ref[...] = jnp.broadcast_to(carry, (8, E)).astype(jnp.int32)


def _tilemap_kernel(cnt_ref, tb_ref, te_ref, trs_ref, tre_ref):
    BIG = jnp.int32(10 ** 6)

    def emit(t, st):
        bp, gp, gsv = st
        bval = jnp.where(bp < NB, bp * B, BIG)
        gval = jnp.where(gp < E, gsv, BIG)
        take_b = bval <= gval
        rs = jnp.where(take_b, bval, gval)
        nbp = jnp.where(take_b, bp + 1, bp)
        ngp = jnp.where(take_b, gp, gp + 1)
        ngsv = jnp.where(take_b, gsv,
                         gsv + cnt_ref[jnp.minimum(ngp - 1, E - 1)])
        trs_ref[t] = rs
        te_ref[t] = ngp - 1
        tb_ref[t] = jnp.minimum(rs // B, NB - 1)
        return nbp, ngp, ngsv

    jax.lax.fori_loop(0, NT, emit, (jnp.int32(0), jnp.int32(1),
                                    cnt_ref[0]))

    def fill_re(t, _):
        tre_ref[t] = jnp.where(t < NT - 1, trs_ref[t + 1], S)
        return 0

    jax.lax.fori_loop(0, NT, fill_re, 0)


def _gmm_kernel(tb_ref, te_ref, trs_ref, tre_ref,
                xg_ref, gp_ref, up_ref, dp_ref, y_ref):
    t = pl.program_id(0)
    rs = trs_ref[t]
    re = tre_ref[t]
    base = tb_ref[t] * B
    rows = base + jax.lax.broadcasted_iota(jnp.int32, (B, 1), 0)
    mask = jnp.logical_and(rows >= rs, rows < re).astype(jnp.float32)

    xgb = xg_ref[...]
    acc = jnp.zeros((B, D), jnp.float32)
    KC = 512
    for kc in range(DFF // KC):
        gp = gp_ref[0, pl.ds(kc * KC, KC), :]
        up = up_ref[0, pl.ds(kc * KC, KC), :]
        h = jax.nn.silu(jax.lax.dot_general(
            xgb, gp, (((1,), (1,)), ((), ())),
            preferred_element_type=jnp.float32))
        h = h * jax.lax.dot_general(
            xgb, up, (((1,), (1,)), ((), ())),
            preferred_element_type=jnp.float32)
        dp = dp_ref[0, :, pl.ds(kc * KC, KC)]
        acc = acc + jax.lax.dot_general(
            h.astype(jnp.bfloat16), dp, (((1,), (1,)), ((), ())),
            preferred_element_type=jnp.float32)
    contrib = acc * mask

    prev = tb_ref[jnp.maximum(t - 1, 0)]
    new_blk = jnp.logical_or(t == 0, tb_ref[t] != prev)

    @pl.when(new_blk)
    def _init():
        y_ref[...] = contrib

    @pl.when(jnp.logical_not(new_blk))
    def _acc():
        y_ref[...] += contrib


def _route(gating_output):
    dest8, w8, cnt8 = pl.pallas_call(
        _route_kernel,
        out_shape=[
            jax.ShapeDtypeStruct((S, E), jnp.int32),
            jax.ShapeDtypeStruct((S, E), jnp.float32),
            jax.ShapeDtypeStruct((8, E), jnp.int32),
        ],
    )(gating_output)
    counts = cnt8[0]
    tb, te, trs, tre = pl.pallas_call(
        _tilemap_kernel,
        in_specs=[pl.BlockSpec(memory_space=pltpu.SMEM)],
        out_specs=[pl.BlockSpec(memory_space=pltpu.SMEM)] * 4,
        out_shape=[jax.ShapeDtypeStruct((NT,), jnp.int32)] * 4,
    )(counts)
    return dest8[:, 0], w8[:, 0], tb, te, trs, tre


def _gmm(xg, gate_b, up_b, down_b, tb, te, trs, tre):
    grid_spec = pltpu.PrefetchScalarGridSpec(
        num_scalar_prefetch=4,
        grid=(NT,),
        in_specs=[
            pl.BlockSpec((B, D), lambda t, tb, te, trs, tre: (tb[t], 0)),
            pl.BlockSpec((1, DFF, D), lambda t, tb, te, trs, tre: (te[t], 0, 0)),
            pl.BlockSpec((1, DFF, D), lambda t, tb, te, trs, tre: (te[t], 0, 0)),
            pl.BlockSpec((1, D, DFF), lambda t, tb, te, trs, tre: (te[t], 0, 0)),
        ],
        out_specs=pl.BlockSpec((B, D), lambda t, tb, te, trs, tre: (tb[t], 0)),
    )
    return pl.pallas_call(
        _gmm_kernel,
        grid_spec=grid_spec,
        out_shape=jax.ShapeDtypeStruct((S, D), jnp.float32),
        compiler_params=pltpu.CompilerParams(
            dimension_semantics=("arbitrary",),
        ),
    )(tb, te, trs, tre, xg, gate_b, up_b, down_b)


NW = 32
_SC_MESH = dict(core_axis_name="c", subcore_axis_name="s")
K2C = 32
K4C = 16


def _wid():
    return lax.axis_index("s") * 2 + lax.axis_index("c")


@functools.partial(
    pl.kernel,
    mesh=plsc.VectorSubcoreMesh(**_SC_MESH),
    out_type=jax.ShapeDtypeStruct((S, D // 2), jnp.int32),
    scratch_types=[
        pltpu.VMEM((K2C,), jnp.int32),
        pltpu.VMEM((K2C, D // 2), jnp.int32),
        pltpu.SemaphoreType.DMA,
    ],
)
def _sc_scatter(x_hbm, dest_hbm, xg_hbm, idx_v, rows_v, sem):
    wid = _wid()
    spw = S // NW
    sbase = wid * spw
    xbase = lax.rem(sbase, T)

    def chunk(c, _):
        off = c * K2C
        pltpu.sync_copy(dest_hbm.at[pl.ds(sbase + off, K2C)], idx_v)
        pltpu.sync_copy(x_hbm.at[pl.ds(xbase + off, K2C)], rows_v)
        pltpu.async_copy(rows_v, xg_hbm.at[idx_v], sem).wait()
        return 0

    lax.fori_loop(0, spw // K2C, chunk, 0)


@functools.partial(
    pl.kernel,
    mesh=plsc.VectorSubcoreMesh(**_SC_MESH),
    out_type=jax.ShapeDtypeStruct((T, D), jnp.float32),
    scratch_types=[
        pltpu.VMEM((K4C,), jnp.int32),
        pltpu.VMEM((K4C,), jnp.int32),
        pltpu.VMEM((K4C, D), jnp.float32),
        pltpu.VMEM((K4C, D), jnp.float32),
        pltpu.VMEM((K4C, 16), jnp.float32),
        pltpu.VMEM((K4C, 16), jnp.float32),
        pltpu.VMEM((K4C, D), jnp.float32),
        pltpu.SemaphoreType.DMA,
    ],
)
def _sc_combine(y_hbm, dest_hbm, w16_hbm, out_hbm,
                idx0_v, idx1_v, y0_v, y1_v, w0_v, w1_v, ob_v, sem):
    wid = _wid()
    tpw = T // NW
    tbase = wid * tpw
    NCOL = D // 16

    def chunk(c, _):
        t0 = tbase + c * K4C
        pltpu.sync_copy(dest_hbm.at[pl.ds(t0, K4C)], idx0_v)
        pltpu.sync_copy(dest_hbm.at[pl.ds(T + t0, K4C)], idx1_v)
        pltpu.sync_copy(w16_hbm.at[pl.ds(t0, K4C)], w0_v)
        pltpu.sync_copy(w16_hbm.at[pl.ds(T + t0, K4C)], w1_v)
        pltpu.async_copy(y_hbm.at[idx0_v], y0_v, sem).wait()
        pltpu.async_copy(y_hbm.at[idx1_v], y1_v, sem).wait()

        def row(j, _):
            w0 = w0_v[j, :]
            w1 = w1_v[j, :]

            def col(cc, _):
                sl = pl.ds(cc * 16, 16)
                ob_v[j, sl] = w0 * y0_v[j, sl] + w1 * y1_v[j, sl]
                return 0

            lax.fori_loop(0, NCOL, col, 0)
            return 0

        lax.fori_loop(0, K4C, row, 0)
        pltpu.sync_copy(ob_v, out_hbm.at[pl.ds(t0, K4C)])
        return 0

    lax.fori_loop(0, tpw // K4C, chunk, 0)


@functools.partial(jax.jit, static_argnums=())
def kernel(x, gating_output, gate_proj, up_proj, down_proj):
    dest, w_flat, tb, te, trs, tre = _route(gating_output)
    gate_b = gate_proj.astype(jnp.bfloat16)
    up_b = up_proj.astype(jnp.bfloat16)
    down_b = down_proj.astype(jnp.bfloat16)
    w16 = jnp.broadcast_to(w_flat[:, None], (S, 16))

    x32 = jax.lax.bitcast_convert_type(
        x.astype(jnp.bfloat16).reshape(T, D // 2, 2), jnp.int32)
    xg32 = _sc_scatter(x32, dest)
    xg = jax.lax.bitcast_convert_type(xg32, jnp.bfloat16).reshape(S, D)
    y = _gmm(xg, gate_b, up_b, down_b, tb, te, trs, tre)
    out = _sc_combine(y, dest, w16)
    return out


# speedup vs baseline: 1.2221x; 1.2221x over previous
import functools

import jax
import jax.numpy as jnp
from jax import lax
from jax.experimental import pallas as pl
from jax.experimental.pallas import tpu as pltpu
from jax.experimental.pallas import tpu_sc as plsc

T = 2048
D = 1024
E = 8
DFF = 2048
S = 2 * T
B = 256
NB = S // B
NT = NB + E - 1
CH = 512


def _route_kernel(g_ref, dest_ref, w_ref, cnt_ref):
    g = g_ref[...].astype(jnp.float32)
    m = jnp.max(g, axis=-1, keepdims=True)
    p = jnp.exp(g - m)
    scores = p / jnp.sum(p, axis=-1, keepdims=True)
    i1 = jnp.argmax(scores, axis=-1)[:, None]
    lanes_t = jax.lax.broadcasted_iota(jnp.int32, (T, E), 1)
    one1 = lanes_t == i1
    w1 = jnp.max(scores, axis=-1, keepdims=True)
    masked = jnp.where(one1, -jnp.inf, scores)
    i2 = jnp.argmax(masked, axis=-1)[:, None]
    w2 = jnp.max(masked, axis=-1, keepdims=True)
    denom = w1 + w2
    w1n, w2n = w1 / denom, w2 / denom

    e_flat = jnp.concatenate([i1, i2], axis=0)
    w_flat = jnp.concatenate([w1n, w2n], axis=0)

    lanes = jax.lax.broadcasted_iota(jnp.int32, (S, E), 1)
    onehot = (lanes == e_flat).astype(jnp.float32)

    r = jax.lax.broadcasted_iota(jnp.int32, (CH, CH), 0)
    c = jax.lax.broadcasted_iota(jnp.int32, (CH, CH), 1)
    tril = (r >= c).astype(jnp.float32)

    carry = jnp.zeros((1, E), jnp.float32)
    blocks = []
    for ci in range(S // CH):
        blk = jax.lax.slice(onehot, (ci * CH, 0), ((ci + 1) * CH, E))
        incl = jax.lax.dot_general(
            tril, blk, (((1,), (0,)), ((), ())),
            preferred_element_type=jnp.float32) + carry
        carry = jax.lax.slice(incl, (CH - 1, 0), (CH, E))
        blocks.append(incl)
    incl_all = jnp.concatenate(blocks, axis=0)

    counts = jnp.broadcast_to(carry, (S, E))
    before = jnp.sum(jnp.where(lanes < e_flat, counts, 0.0), axis=1,
                     keepdims=True)
    own = jnp.sum(jnp.where(lanes == e_flat, incl_all, 0.0), axis=1,
                  keepdims=True)
    dest = (before + own - 1.0).astype(jnp.int32)

    dest_ref[...] = jnp.broadcast_to(dest, (S, E))
    w_ref[...] = jnp.broadcast_to(w_flat, (S, E))
    cnt_ref[...] = jnp.broadcast_to(carry, (8, E)).astype(jnp.int32)


def _tilemap_kernel(cnt_ref, tb_ref, te_ref, trs_ref, tre_ref):
    BIG = jnp.int32(10 ** 6)

    def emit(t, st):
        bp, gp, gsv = st
        bval = jnp.where(bp < NB, bp * B, BIG)
        gval = jnp.where(gp < E, gsv, BIG)
        take_b = bval <= gval
        rs = jnp.where(take_b, bval, gval)
        nbp = jnp.where(take_b, bp + 1, bp)
        ngp = jnp.where(take_b, gp, gp + 1)
        ngsv = jnp.where(take_b, gsv,
                         gsv + cnt_ref[jnp.minimum(ngp - 1, E - 1)])
        trs_ref[t] = rs
        te_ref[t] = ngp - 1
        tb_ref[t] = jnp.minimum(rs // B, NB - 1)
        return nbp, ngp, ngsv

    jax.lax.fori_loop(0, NT, emit, (jnp.int32(0), jnp.int32(1),
                                    cnt_ref[0]))

    def fill_re(t, _):
        tre_ref[t] = jnp.where(t < NT - 1, trs_ref[t + 1], S)
        return 0

    jax.lax.fori_loop(0, NT, fill_re, 0)


def _gmm_kernel(tb_ref, te_ref, trs_ref, tre_ref,
                xg_ref, gp_ref, up_ref, dp_ref, y_ref):
    t = pl.program_id(0)
    rs = trs_ref[t]
    re = tre_ref[t]
    base = tb_ref[t] * B
    rows = base + jax.lax.broadcasted_iota(jnp.int32, (B, 1), 0)
    mask = jnp.logical_and(rows >= rs, rows < re).astype(jnp.float32)

    xgb = xg_ref[...]
    acc = jnp.zeros((B, D), jnp.float32)
    KC = 512
    for kc in range(DFF // KC):
        gp = gp_ref[0, pl.ds(kc * KC, KC), :]
        up = up_ref[0, pl.ds(kc * KC, KC), :]
        h = jax.nn.silu(jax.lax.dot_general(
            xgb, gp, (((1,), (1,)), ((), ())),
            preferred_element_type=jnp.float32))
        h = h * jax.lax.dot_general(
            xgb, up, (((1,), (1,)), ((), ())),
            preferred_element_type=jnp.float32)
        dp = dp_ref[0, :, pl.ds(kc * KC, KC)]
        acc = acc + jax.lax.dot_general(
            h.astype(jnp.bfloat16), dp, (((1,), (1,)), ((), ())),
            preferred_element_type=jnp.float32)
    contrib = acc * mask

    prev = tb_ref[jnp.maximum(t - 1, 0)]
    new_blk = jnp.logical_or(t == 0, tb_ref[t] != prev)

    @pl.when(new_blk)
    def _init():
        y_ref[...] = contrib

    @pl.when(jnp.logical_not(new_blk))
    def _acc():
        y_ref[...] += contrib


def _route(gating_output):
    dest8, w8, cnt8 = pl.pallas_call(
        _route_kernel,
        out_shape=[
            jax.ShapeDtypeStruct((S, E), jnp.int32),
            jax.ShapeDtypeStruct((S, E), jnp.float32),
            jax.ShapeDtypeStruct((8, E), jnp.int32),
        ],
    )(gating_output)
    counts = cnt8[0]
    tb, te, trs, tre = pl.pallas_call(
        _tilemap_kernel,
        in_specs=[pl.BlockSpec(memory_space=pltpu.SMEM)],
        out_specs=[pl.BlockSpec(memory_space=pltpu.SMEM)] * 4,
        out_shape=[jax.ShapeDtypeStruct((NT,), jnp.int32)] * 4,
    )(counts)
    return dest8[:, 0], w8[:, 0], tb, te, trs, tre


def _gmm(xg, gate_b, up_b, down_b, tb, te, trs, tre):
    grid_spec = pltpu.PrefetchScalarGridSpec(
        num_scalar_prefetch=4,
        grid=(NT,),
        in_specs=[
            pl.BlockSpec((B, D), lambda t, tb, te, trs, tre: (tb[t], 0)),
            pl.BlockSpec((1, DFF, D), lambda t, tb, te, trs, tre: (te[t], 0, 0)),
            pl.BlockSpec((1, DFF, D), lambda t, tb, te, trs, tre: (te[t], 0, 0)),
            pl.BlockSpec((1, D, DFF), lambda t, tb, te, trs, tre: (te[t], 0, 0)),
        ],
        out_specs=pl.BlockSpec((B, D), lambda t, tb, te, trs, tre: (tb[t], 0)),
    )
    return pl.pallas_call(
        _gmm_kernel,
        grid_spec=grid_spec,
        out_shape=jax.ShapeDtypeStruct((S, D), jnp.float32),
        compiler_params=pltpu.CompilerParams(
            dimension_semantics=("arbitrary",),
        ),
    )(tb, te, trs, tre, xg, gate_b, up_b, down_b)


NW = 32
_SC_MESH = dict(core_axis_name="c", subcore_axis_name="s")
K2C = 32
K4C = 16


def _wid():
    return lax.axis_index("s") * 2 + lax.axis_index("c")


@functools.partial(
    pl.kernel,
    mesh=plsc.VectorSubcoreMesh(**_SC_MESH),
    out_type=jax.ShapeDtypeStruct((S, D // 2), jnp.int32),
    scratch_types=[
        pltpu.VMEM((K2C,), jnp.int32),
        pltpu.VMEM((K2C, D // 2), jnp.int32),
        pltpu.SemaphoreType.DMA,
    ],
)
def _sc_scatter(x_hbm, dest_hbm, xg_hbm, idx_v, rows_v, sem):
    wid = _wid()
    spw = S // NW
    sbase = wid * spw
    xbase = lax.rem(sbase, T)

    def chunk(c, _):
        off = c * K2C
        pltpu.sync_copy(dest_hbm.at[pl.ds(sbase + off, K2C)], idx_v)
        pltpu.sync_copy(x_hbm.at[pl.ds(xbase + off, K2C)], rows_v)
        pltpu.async_copy(rows_v, xg_hbm.at[idx_v], sem).wait()
        return 0

    lax.fori_loop(0, spw // K2C, chunk, 0)


@functools.partial(
    pl.kernel,
    mesh=plsc.VectorSubcoreMesh(**_SC_MESH),
    out_type=jax.ShapeDtypeStruct((T, D), jnp.float32),
    scratch_types=[
        pltpu.VMEM((K4C,), jnp.int32),
        pltpu.VMEM((K4C,), jnp.int32),
        pltpu.VMEM((K4C, D), jnp.float32),
        pltpu.VMEM((K4C, D), jnp.float32),
        pltpu.VMEM((K4C, 16), jnp.float32),
        pltpu.VMEM((K4C, 16), jnp.float32),
        pltpu.VMEM((K4C, D), jnp.float32),
        pltpu.SemaphoreType.DMA,
    ],
)
def _sc_combine(y_hbm, dest_hbm, w16_hbm, out_hbm,
                idx0_v, idx1_v, y0_v, y1_v, w0_v, w1_v, ob_v, sem):
    wid = _wid()
    tpw = T // NW
    tbase = wid * tpw
    NCOL = D // 16

    def chunk(c, _):
        t0 = tbase + c * K4C
        pltpu.sync_copy(dest_hbm.at[pl.ds(t0, K4C)], idx0_v)
        pltpu.sync_copy(dest_hbm.at[pl.ds(T + t0, K4C)], idx1_v)
        pltpu.sync_copy(w16_hbm.at[pl.ds(t0, K4C)], w0_v)
        pltpu.sync_copy(w16_hbm.at[pl.ds(T + t0, K4C)], w1_v)
        pltpu.async_copy(y_hbm.at[idx0_v], y0_v, sem).wait()
        pltpu.async_copy(y_hbm.at[idx1_v], y1_v, sem).wait()

        def row(j, _):
            w0 = w0_v[j, :]
            w1 = w1_v[j, :]

            def col(cc, _):
                sl = pl.ds(cc * 16, 16)
                ob_v[j, sl] = w0 * y0_v[j, sl] + w1 * y1_v[j, sl]
                return 0

            lax.fori_loop(0, NCOL, col, 0)
            return 0

        lax.fori_loop(0, K4C, row, 0)
        pltpu.sync_copy(ob_v, out_hbm.at[pl.ds(t0, K4C)])
        return 0

    lax.fori_loop(0, tpw // K4C, chunk, 0)


@functools.partial(jax.jit, static_argnums=())
def kernel(x, gating_output, gate_proj, up_proj, down_proj):
    dest, w_flat, tb, te, trs, tre = _route(gating_output)
    gate_b = gate_proj.astype(jnp.bfloat16)
    up_b = up_proj.astype(jnp.bfloat16)
    down_b = down_proj.astype(jnp.bfloat16)
    w16 = jnp.broadcast_to(w_flat[:, None], (S, 16))

    x32 = jax.lax.bitcast_convert_type(
        x.astype(jnp.bfloat16).reshape(T, D // 2, 2), jnp.int32)
    xg32 = _sc_scatter(x32, dest)
    xg = jax.lax.bitcast_convert_type(xg32, jnp.bfloat16).reshape(S, D)
    y = _gmm(xg, gate_b, up_b, down_b, tb, te, trs, tre)
    out = _sc_combine(y, dest, w16)
    return out


# speedup vs baseline: 1.7112x; 1.4002x over previous
import functools

import jax
import jax.numpy as jnp
from jax import lax
from jax.experimental import pallas as pl
from jax.experimental.pallas import tpu as pltpu
from jax.experimental.pallas import tpu_sc as plsc

T = 2048
D = 1024
E = 8
DFF = 2048
S = 2 * T
B = 256
NB = S // B
NT = NB + E - 1
CH = 512


def _route_kernel(g_ref, dest_ref, w_ref, cnt_ref):
    g = g_ref[...].astype(jnp.float32)
    m = jnp.max(g, axis=-1, keepdims=True)
    p = jnp.exp(g - m)
    scores = p / jnp.sum(p, axis=-1, keepdims=True)
    i1 = jnp.argmax(scores, axis=-1)[:, None]
    lanes_t = jax.lax.broadcasted_iota(jnp.int32, (T, E), 1)
    one1 = lanes_t == i1
    w1 = jnp.max(scores, axis=-1, keepdims=True)
    masked = jnp.where(one1, -jnp.inf, scores)
    i2 = jnp.argmax(masked, axis=-1)[:, None]
    w2 = jnp.max(masked, axis=-1, keepdims=True)
    denom = w1 + w2
    w1n, w2n = w1 / denom, w2 / denom

    e_flat = jnp.concatenate([i1, i2], axis=0)
    w_flat = jnp.concatenate([w1n, w2n], axis=0)

    lanes = jax.lax.broadcasted_iota(jnp.int32, (S, E), 1)
    onehot = (lanes == e_flat).astype(jnp.float32)

    r = jax.lax.broadcasted_iota(jnp.int32, (CH, CH), 0)
    c = jax.lax.broadcasted_iota(jnp.int32, (CH, CH), 1)
    tril = (r >= c).astype(jnp.float32)

    carry = jnp.zeros((1, E), jnp.float32)
    blocks = []
    for ci in range(S // CH):
        blk = jax.lax.slice(onehot, (ci * CH, 0), ((ci + 1) * CH, E))
        incl = jax.lax.dot_general(
            tril, blk, (((1,), (0,)), ((), ())),
            preferred_element_type=jnp.float32) + carry
        carry = jax.lax.slice(incl, (CH - 1, 0), (CH, E))
        blocks.append(incl)
    incl_all = jnp.concatenate(blocks, axis=0)

    counts = jnp.broadcast_to(carry, (S, E))
    before = jnp.sum(jnp.where(lanes < e_flat, counts, 0.0), axis=1,
                     keepdims=True)
    own = jnp.sum(jnp.where(lanes == e_flat, incl_all, 0.0), axis=1,
                  keepdims=True)
    dest = (before + own - 1.0).astype(jnp.int32)

    dest_ref[...] = jnp.broadcast_to(dest, (S, E))
    w_ref[...] = jnp.broadcast_to(w_flat, (S, E))
    cnt_ref[...] = jnp.broadcast_to(carry, (8, E)).astype(jnp.int32)


def _tilemap_kernel(cnt_ref, tb_ref, te_ref, trs_ref, tre_ref):
    BIG = jnp.int32(10 ** 6)

    def emit(t, st):
        bp, gp, gsv = st
        bval = jnp.where(bp < NB, bp * B, BIG)
        gval = jnp.where(gp < E, gsv, BIG)
        take_b = bval <= gval
        rs = jnp.where(take_b, bval, gval)
        nbp = jnp.where(take_b, bp + 1, bp)
        ngp = jnp.where(take_b, gp, gp + 1)
        ngsv = jnp.where(take_b, gsv,
                         gsv + cnt_ref[jnp.minimum(ngp - 1, E - 1)])
        trs_ref[t] = rs
        te_ref[t] = ngp - 1
        tb_ref[t] = jnp.minimum(rs // B, NB - 1)
        return nbp, ngp, ngsv

    jax.lax.fori_loop(0, NT, emit, (jnp.int32(0), jnp.int32(1),
                                    cnt_ref[0]))

    def fill_re(t, _):
        tre_ref[t] = jnp.where(t < NT - 1, trs_ref[t + 1], S)
        return 0

    jax.lax.fori_loop(0, NT, fill_re, 0)


def _gmm_kernel(tb_ref, te_ref, trs_ref, tre_ref,
                xg_ref, gp_ref, up_ref, dp_ref, y_ref):
    t = pl.program_id(0)
    rs = trs_ref[t]
    re = tre_ref[t]
    base = tb_ref[t] * B
    rows = base + jax.lax.broadcasted_iota(jnp.int32, (B, 1), 0)
    mask = jnp.logical_and(rows >= rs, rows < re).astype(jnp.float32)

    xgb = xg_ref[...].astype(jnp.bfloat16)
    acc = jnp.zeros((B, D), jnp.float32)
    KC = 512
    for kc in range(DFF // KC):
        gp = gp_ref[0, pl.ds(kc * KC, KC), :]
        up = up_ref[0, pl.ds(kc * KC, KC), :]
        h = jax.nn.silu(jax.lax.dot_general(
            xgb, gp, (((1,), (1,)), ((), ())),
            preferred_element_type=jnp.float32))
        h = h * jax.lax.dot_general(
            xgb, up, (((1,), (1,)), ((), ())),
            preferred_element_type=jnp.float32)
        dp = dp_ref[0, :, pl.ds(kc * KC, KC)]
        acc = acc + jax.lax.dot_general(
            h.astype(jnp.bfloat16), dp, (((1,), (1,)), ((), ())),
            preferred_element_type=jnp.float32)
    contrib = acc * mask

    prev = tb_ref[jnp.maximum(t - 1, 0)]
    new_blk = jnp.logical_or(t == 0, tb_ref[t] != prev)

    @pl.when(new_blk)
    def _init():
        y_ref[...] = contrib

    @pl.when(jnp.logical_not(new_blk))
    def _acc():
        y_ref[...] += contrib


def _route(gating_output):
    dest8, w8, cnt8 = pl.pallas_call(
        _route_kernel,
        out_shape=[
            jax.ShapeDtypeStruct((S, E), jnp.int32),
            jax.ShapeDtypeStruct((S, E), jnp.float32),
            jax.ShapeDtypeStruct((8, E), jnp.int32),
        ],
    )(gating_output)
    counts = cnt8[0]
    tb, te, trs, tre = pl.pallas_call(
        _tilemap_kernel,
        in_specs=[pl.BlockSpec(memory_space=pltpu.SMEM)],
        out_specs=[pl.BlockSpec(memory_space=pltpu.SMEM)] * 4,
        out_shape=[jax.ShapeDtypeStruct((NT,), jnp.int32)] * 4,
    )(counts)
    return dest8[:, 0], w8[:, 0], tb, te, trs, tre


def _gmm(xg, gate_b, up_b, down_b, tb, te, trs, tre):
    grid_spec = pltpu.PrefetchScalarGridSpec(
        num_scalar_prefetch=4,
        grid=(NT,),
        in_specs=[
            pl.BlockSpec((B, D), lambda t, tb, te, trs, tre: (tb[t], 0)),
            pl.BlockSpec((1, DFF, D), lambda t, tb, te, trs, tre: (te[t], 0, 0)),
            pl.BlockSpec((1, DFF, D), lambda t, tb, te, trs, tre: (te[t], 0, 0)),
            pl.BlockSpec((1, D, DFF), lambda t, tb, te, trs, tre: (te[t], 0, 0)),
        ],
        out_specs=pl.BlockSpec((B, D), lambda t, tb, te, trs, tre: (tb[t], 0)),
    )
    return pl.pallas_call(
        _gmm_kernel,
        grid_spec=grid_spec,
        out_shape=jax.ShapeDtypeStruct((S, D), jnp.float32),
        compiler_params=pltpu.CompilerParams(
            dimension_semantics=("arbitrary",),
        ),
    )(tb, te, trs, tre, xg, gate_b, up_b, down_b)


NW = 32
_SC_MESH = dict(core_axis_name="c", subcore_axis_name="s")
K2C = 32
K4C = 16


def _wid():
    return lax.axis_index("s") * 2 + lax.axis_index("c")


@functools.partial(
    pl.kernel,
    mesh=plsc.VectorSubcoreMesh(**_SC_MESH),
    out_type=jax.ShapeDtypeStruct((S, D), jnp.float32),
    scratch_types=[
        pltpu.VMEM((K2C,), jnp.int32),
        pltpu.VMEM((K2C, D), jnp.float32),
        pltpu.SemaphoreType.DMA,
    ],
)
def _sc_scatter(x_hbm, dest_hbm, xg_hbm, idx_v, rows_v, sem):
    wid = _wid()
    spw = S // NW
    sbase = wid * spw
    xbase = lax.rem(sbase, T)

    def chunk(c, _):
        off = c * K2C
        pltpu.sync_copy(dest_hbm.at[pl.ds(sbase + off, K2C)], idx_v)
        pltpu.sync_copy(x_hbm.at[pl.ds(xbase + off, K2C)], rows_v)
        pltpu.async_copy(rows_v, xg_hbm.at[idx_v], sem).wait()
        return 0

    lax.fori_loop(0, spw // K2C, chunk, 0)


@functools.partial(
    pl.kernel,
    mesh=plsc.VectorSubcoreMesh(**_SC_MESH),
    out_type=jax.ShapeDtypeStruct((T, D), jnp.float32),
    scratch_types=[
        pltpu.VMEM((K4C,), jnp.int32),
        pltpu.VMEM((K4C,), jnp.int32),
        pltpu.VMEM((K4C, D), jnp.float32),
        pltpu.VMEM((K4C, D), jnp.float32),
        pltpu.VMEM((K4C, 16), jnp.float32),
        pltpu.VMEM((K4C, 16), jnp.float32),
        pltpu.VMEM((K4C, D), jnp.float32),
        pltpu.SemaphoreType.DMA,
    ],
)
def _sc_combine(y_hbm, dest_hbm, w16_hbm, out_hbm,
                idx0_v, idx1_v, y0_v, y1_v, w0_v, w1_v, ob_v, sem):
    wid = _wid()
    tpw = T // NW
    tbase = wid * tpw
    NCOL = D // 16

    def chunk(c, _):
        t0 = tbase + c * K4C
        pltpu.sync_copy(dest_hbm.at[pl.ds(t0, K4C)], idx0_v)
        pltpu.sync_copy(dest_hbm.at[pl.ds(T + t0, K4C)], idx1_v)
        pltpu.sync_copy(w16_hbm.at[pl.ds(t0, K4C)], w0_v)
        pltpu.sync_copy(w16_hbm.at[pl.ds(T + t0, K4C)], w1_v)
        pltpu.async_copy(y_hbm.at[idx0_v], y0_v, sem).wait()
        pltpu.async_copy(y_hbm.at[idx1_v], y1_v, sem).wait()

        def row(j, _):
            w0 = w0_v[j, :]
            w1 = w1_v[j, :]

            def col(cc, _):
                sl = pl.ds(cc * 16, 16)
                ob_v[j, sl] = w0 * y0_v[j, sl] + w1 * y1_v[j, sl]
                return 0

            lax.fori_loop(0, NCOL, col, 0)
            return 0

        lax.fori_loop(0, K4C, row, 0)
        pltpu.sync_copy(ob_v, out_hbm.at[pl.ds(t0, K4C)])
        return 0

    lax.fori_loop(0, tpw // K4C, chunk, 0)


@functools.partial(jax.jit, static_argnums=())
def kernel(x, gating_output, gate_proj, up_proj, down_proj):
    dest, w_flat, tb, te, trs, tre = _route(gating_output)
    gate_b = gate_proj.astype(jnp.bfloat16)
    up_b = up_proj.astype(jnp.bfloat16)
    down_b = down_proj.astype(jnp.bfloat16)
    w16 = jnp.broadcast_to(w_flat[:, None], (S, 16))

    xg = _sc_scatter(x, dest)
    y = _gmm(xg, gate_b, up_b, down_b, tb, te, trs, tre)
    out = _sc_combine(y, dest, w16)
    return out


# speedup vs baseline: 1.7672x; 1.0327x over previous
import functools

import jax
import jax.numpy as jnp
from jax import lax
from jax.experimental import pallas as pl
from jax.experimental.pallas import tpu as pltpu
from jax.experimental.pallas import tpu_sc as plsc

T = 2048
D = 1024
E = 8
DFF = 2048
S = 2 * T
B = 256
NB = S // B
NT = NB + E - 1
CH = 512


def _route_kernel(g_ref, dest_ref, w_ref, cnt_ref):
    g = g_ref[...].astype(jnp.float32)
    m = jnp.max(g, axis=-1, keepdims=True)
    p = jnp.exp(g - m)
    scores = p / jnp.sum(p, axis=-1, keepdims=True)
    i1 = jnp.argmax(scores, axis=-1)[:, None]
    lanes_t = jax.lax.broadcasted_iota(jnp.int32, (T, E), 1)
    one1 = lanes_t == i1
    w1 = jnp.max(scores, axis=-1, keepdims=True)
    masked = jnp.where(one1, -jnp.inf, scores)
    i2 = jnp.argmax(masked, axis=-1)[:, None]
    w2 = jnp.max(masked, axis=-1, keepdims=True)
    denom = w1 + w2
    w1n, w2n = w1 / denom, w2 / denom

    e_flat = jnp.concatenate([i1, i2], axis=0)
    w_flat = jnp.concatenate([w1n, w2n], axis=0)

    lanes = jax.lax.broadcasted_iota(jnp.int32, (S, E), 1)
    onehot = (lanes == e_flat).astype(jnp.float32)

    r = jax.lax.broadcasted_iota(jnp.int32, (CH, CH), 0)
    c = jax.lax.broadcasted_iota(jnp.int32, (CH, CH), 1)
    tril = (r >= c).astype(jnp.float32)

    carry = jnp.zeros((1, E), jnp.float32)
    blocks = []
    for ci in range(S // CH):
        blk = jax.lax.slice(onehot, (ci * CH, 0), ((ci + 1) * CH, E))
        incl = jax.lax.dot_general(
            tril, blk, (((1,), (0,)), ((), ())),
            preferred_element_type=jnp.float32) + carry
        carry = jax.lax.slice(incl, (CH - 1, 0), (CH, E))
        blocks.append(incl)
    incl_all = jnp.concatenate(blocks, axis=0)

    counts = jnp.broadcast_to(carry, (S, E))
    before = jnp.sum(jnp.where(lanes < e_flat, counts, 0.0), axis=1,
                     keepdims=True)
    own = jnp.sum(jnp.where(lanes == e_flat, incl_all, 0.0), axis=1,
                  keepdims=True)
    dest = (before + own - 1.0).astype(jnp.int32)

    dest_ref[...] = jnp.broadcast_to(dest, (S, E))
    w_ref[...] = jnp.broadcast_to(w_flat, (S, E))
    cnt_ref[...] = jnp.broadcast_to(carry, (8, E)).astype(jnp.int32)


def _tilemap_kernel(cnt_ref, tb_ref, te_ref, trs_ref, tre_ref):
    BIG = jnp.int32(10 ** 6)

    def emit(t, st):
        bp, gp, gsv = st
        bval = jnp.where(bp < NB, bp * B, BIG)
        gval = jnp.where(gp < E, gsv, BIG)
        take_b = bval <= gval
        rs = jnp.where(take_b, bval, gval)
        nbp = jnp.where(take_b, bp + 1, bp)
        ngp = jnp.where(take_b, gp, gp + 1)
        ngsv = jnp.where(take_b, gsv,
                         gsv + cnt_ref[jnp.minimum(ngp - 1, E - 1)])
        trs_ref[t] = rs
        te_ref[t] = ngp - 1
        tb_ref[t] = jnp.minimum(rs // B, NB - 1)
        return nbp, ngp, ngsv

    jax.lax.fori_loop(0, NT, emit, (jnp.int32(0), jnp.int32(1),
                                    cnt_ref[0]))

    def fill_re(t, _):
        tre_ref[t] = jnp.where(t < NT - 1, trs_ref[t + 1], S)
        return 0

    jax.lax.fori_loop(0, NT, fill_re, 0)


def _gmm_kernel(tb_ref, te_ref, trs_ref, tre_ref,
                xg_ref, gp_ref, up_ref, dp_ref, y_ref):
    t = pl.program_id(0)
    rs = trs_ref[t]
    re = tre_ref[t]
    base = tb_ref[t] * B
    rows = base + jax.lax.broadcasted_iota(jnp.int32, (B, 1), 0)
    mask = jnp.logical_and(rows >= rs, rows < re).astype(jnp.float32)

    xgb = xg_ref[...].astype(jnp.bfloat16)
    acc = jnp.zeros((B, D), jnp.float32)
    KC = 512
    for kc in range(DFF // KC):
        gp = gp_ref[0, pl.ds(kc * KC, KC), :]
        up = up_ref[0, pl.ds(kc * KC, KC), :]
        h = jax.nn.silu(jax.lax.dot_general(
            xgb, gp, (((1,), (1,)), ((), ())),
            preferred_element_type=jnp.float32))
        h = h * jax.lax.dot_general(
            xgb, up, (((1,), (1,)), ((), ())),
            preferred_element_type=jnp.float32)
        dp = dp_ref[0, :, pl.ds(kc * KC, KC)]
        acc = acc + jax.lax.dot_general(
            h.astype(jnp.bfloat16), dp, (((1,), (1,)), ((), ())),
            preferred_element_type=jnp.float32)
    contrib = acc * mask

    prev = tb_ref[jnp.maximum(t - 1, 0)]
    new_blk = jnp.logical_or(t == 0, tb_ref[t] != prev)

    @pl.when(new_blk)
    def _init():
        y_ref[...] = contrib

    @pl.when(jnp.logical_not(new_blk))
    def _acc():
        y_ref[...] += contrib


def _route(gating_output):
    dest8, w8, cnt8 = pl.pallas_call(
        _route_kernel,
        out_shape=[
            jax.ShapeDtypeStruct((S, E), jnp.int32),
            jax.ShapeDtypeStruct((S, E), jnp.float32),
            jax.ShapeDtypeStruct((8, E), jnp.int32),
        ],
    )(gating_output)
    counts = cnt8[0]
    tb, te, trs, tre = pl.pallas_call(
        _tilemap_kernel,
        in_specs=[pl.BlockSpec(memory_space=pltpu.SMEM)],
        out_specs=[pl.BlockSpec(memory_space=pltpu.SMEM)] * 4,
        out_shape=[jax.ShapeDtypeStruct((NT,), jnp.int32)] * 4,
    )(counts)
    return dest8[:, 0], w8[:, 0], tb, te, trs, tre


def _gmm(xg, gate_b, up_b, down_b, tb, te, trs, tre):
    grid_spec = pltpu.PrefetchScalarGridSpec(
        num_scalar_prefetch=4,
        grid=(NT,),
        in_specs=[
            pl.BlockSpec((B, D), lambda t, tb, te, trs, tre: (tb[t], 0)),
            pl.BlockSpec((1, DFF, D), lambda t, tb, te, trs, tre: (te[t], 0, 0)),
            pl.BlockSpec((1, DFF, D), lambda t, tb, te, trs, tre: (te[t], 0, 0)),
            pl.BlockSpec((1, D, DFF), lambda t, tb, te, trs, tre: (te[t], 0, 0)),
        ],
        out_specs=pl.BlockSpec((B, D), lambda t, tb, te, trs, tre: (tb[t], 0)),
    )
    return pl.pallas_call(
        _gmm_kernel,
        grid_spec=grid_spec,
        out_shape=jax.ShapeDtypeStruct((S, D), jnp.float32),
        compiler_params=pltpu.CompilerParams(
            dimension_semantics=("arbitrary",),
        ),
    )(tb, te, trs, tre, xg, gate_b, up_b, down_b)


NW = 32
_SC_MESH = dict(core_axis_name="c", subcore_axis_name="s")
K2C = 64
K4C = 32


def _wid():
    return lax.axis_index("s") * 2 + lax.axis_index("c")


@functools.partial(
    pl.kernel,
    mesh=plsc.VectorSubcoreMesh(**_SC_MESH),
    out_type=jax.ShapeDtypeStruct((S, D), jnp.float32),
    scratch_types=[
        pltpu.VMEM((K2C,), jnp.int32),
        pltpu.VMEM((K2C, D), jnp.float32),
        pltpu.SemaphoreType.DMA,
    ],
)
def _sc_scatter(x_hbm, dest_hbm, xg_hbm, idx_v, rows_v, sem):
    wid = _wid()
    spw = S // NW
    sbase = wid * spw
    xbase = lax.rem(sbase, T)

    def chunk(c, _):
        off = c * K2C
        pltpu.sync_copy(dest_hbm.at[pl.ds(sbase + off, K2C)], idx_v)
        pltpu.sync_copy(x_hbm.at[pl.ds(xbase + off, K2C)], rows_v)
        pltpu.async_copy(rows_v, xg_hbm.at[idx_v], sem).wait()
        return 0

    lax.fori_loop(0, spw // K2C, chunk, 0)


@functools.partial(
    pl.kernel,
    mesh=plsc.VectorSubcoreMesh(**_SC_MESH),
    out_type=jax.ShapeDtypeStruct((T, D), jnp.float32),
    scratch_types=[
        pltpu.VMEM((K4C,), jnp.int32),
        pltpu.VMEM((K4C,), jnp.int32),
        pltpu.VMEM((K4C, D), jnp.float32),
        pltpu.VMEM((K4C, D), jnp.float32),
        pltpu.VMEM((64, 16), jnp.float32),
        pltpu.VMEM((64, 16), jnp.float32),
        pltpu.VMEM((K4C, D), jnp.float32),
        pltpu.SemaphoreType.DMA,
    ],
)
def _sc_combine(y_hbm, dest_hbm, w16_hbm, out_hbm,
                idx0_v, idx1_v, y0_v, y1_v, w0_v, w1_v, ob_v, sem):
    wid = _wid()
    tpw = T // NW
    tbase = wid * tpw
    NCOL = D // 16
    pltpu.sync_copy(w16_hbm.at[pl.ds(tbase, tpw)], w0_v)
    pltpu.sync_copy(w16_hbm.at[pl.ds(T + tbase, tpw)], w1_v)

    def chunk(c, _):
        t0 = tbase + c * K4C
        pltpu.sync_copy(dest_hbm.at[pl.ds(t0, K4C)], idx0_v)
        pltpu.sync_copy(dest_hbm.at[pl.ds(T + t0, K4C)], idx1_v)
        g0 = pltpu.async_copy(y_hbm.at[idx0_v], y0_v, sem)
        g1 = pltpu.async_copy(y_hbm.at[idx1_v], y1_v, sem)
        g0.wait()
        g1.wait()

        def row(j, _):
            jw = c * K4C + j
            w0 = w0_v[jw, :]
            w1 = w1_v[jw, :]

            def col(cc, _):
                sl = pl.ds(cc * 16, 16)
                ob_v[j, sl] = w0 * y0_v[j, sl] + w1 * y1_v[j, sl]
                return 0

            lax.fori_loop(0, NCOL, col, 0)
            return 0

        lax.fori_loop(0, K4C, row, 0)
        pltpu.sync_copy(ob_v, out_hbm.at[pl.ds(t0, K4C)])
        return 0

    lax.fori_loop(0, tpw // K4C, chunk, 0)


@functools.partial(jax.jit, static_argnums=())
def kernel(x, gating_output, gate_proj, up_proj, down_proj):
    dest, w_flat, tb, te, trs, tre = _route(gating_output)
    gate_b = gate_proj.astype(jnp.bfloat16)
    up_b = up_proj.astype(jnp.bfloat16)
    down_b = down_proj.astype(jnp.bfloat16)
    w16 = jnp.broadcast_to(w_flat[:, None], (S, 16))

    xg = _sc_scatter(x, dest)
    y = _gmm(xg, gate_b, up_b, down_b, tb, te, trs, tre)
    out = _sc_combine(y, dest, w16)
    return out


# speedup vs baseline: 1.7958x; 1.0162x over previous
import functools

import jax
import jax.numpy as jnp
from jax import lax
from jax.experimental import pallas as pl
from jax.experimental.pallas import tpu as pltpu
from jax.experimental.pallas import tpu_sc as plsc

T = 2048
D = 1024
E = 8
DFF = 2048
S = 2 * T
B = 256
NB = S // B
NT = NB + E - 1
CH = 512


def _route_kernel(g_ref, dest_ref, w_ref, cnt_ref):
    g = g_ref[...].astype(jnp.float32)
    m = jnp.max(g, axis=-1, keepdims=True)
    p = jnp.exp(g - m)
    scores = p / jnp.sum(p, axis=-1, keepdims=True)
    i1 = jnp.argmax(scores, axis=-1)[:, None]
    lanes_t = jax.lax.broadcasted_iota(jnp.int32, (T, E), 1)
    one1 = lanes_t == i1
    w1 = jnp.max(scores, axis=-1, keepdims=True)
    masked = jnp.where(one1, -jnp.inf, scores)
    i2 = jnp.argmax(masked, axis=-1)[:, None]
    w2 = jnp.max(masked, axis=-1, keepdims=True)
    denom = w1 + w2
    w1n, w2n = w1 / denom, w2 / denom

    e_flat = jnp.concatenate([i1, i2], axis=0)
    w_flat = jnp.concatenate([w1n, w2n], axis=0)

    lanes = jax.lax.broadcasted_iota(jnp.int32, (S, E), 1)
    onehot = (lanes == e_flat).astype(jnp.float32)

    r = jax.lax.broadcasted_iota(jnp.int32, (CH, CH), 0)
    c = jax.lax.broadcasted_iota(jnp.int32, (CH, CH), 1)
    tril = (r >= c).astype(jnp.float32)

    carry = jnp.zeros((1, E), jnp.float32)
    blocks = []
    for ci in range(S // CH):
        blk = jax.lax.slice(onehot, (ci * CH, 0), ((ci + 1) * CH, E))
        incl = jax.lax.dot_general(
            tril, blk, (((1,), (0,)), ((), ())),
            preferred_element_type=jnp.float32) + carry
        carry = jax.lax.slice(incl, (CH - 1, 0), (CH, E))
        blocks.append(incl)
    incl_all = jnp.concatenate(blocks, axis=0)

    counts = jnp.broadcast_to(carry, (S, E))
    before = jnp.sum(jnp.where(lanes < e_flat, counts, 0.0), axis=1,
                     keepdims=True)
    own = jnp.sum(jnp.where(lanes == e_flat, incl_all, 0.0), axis=1,
                  keepdims=True)
    dest = (before + own - 1.0).astype(jnp.int32)

    dest_ref[...] = jnp.broadcast_to(dest, (S, E))
    w_ref[...] = jnp.broadcast_to(w_flat, (S, E))
    cnt_ref[...] = jnp.broadcast_to(carry, (8, E)).astype(jnp.int32)


def _tilemap_kernel(cnt_ref, tb_ref, te_ref, trs_ref, tre_ref):
    BIG = jnp.int32(10 ** 6)

    def emit(t, st):
        bp, gp, gsv = st
        bval = jnp.where(bp < NB, bp * B, BIG)
        gval = jnp.where(gp < E, gsv, BIG)
        take_b = bval <= gval
        rs = jnp.where(take_b, bval, gval)
        nbp = jnp.where(take_b, bp + 1, bp)
        ngp = jnp.where(take_b, gp, gp + 1)
        ngsv = jnp.where(take_b, gsv,
                         gsv + cnt_ref[jnp.minimum(ngp - 1, E - 1)])
        trs_ref[t] = rs
        te_ref[t] = ngp - 1
        tb_ref[t] = jnp.minimum(rs // B, NB - 1)
        return nbp, ngp, ngsv

    jax.lax.fori_loop(0, NT, emit, (jnp.int32(0), jnp.int32(1),
                                    cnt_ref[0]))

    def fill_re(t, _):
        tre_ref[t] = jnp.where(t < NT - 1, trs_ref[t + 1], S)
        return 0

    jax.lax.fori_loop(0, NT, fill_re, 0)


def _gmm_kernel(tb_ref, te_ref, trs_ref, tre_ref,
                xg_ref, gp_ref, up_ref, dp_ref, y_ref):
    t = pl.program_id(0)
    rs = trs_ref[t]
    re = tre_ref[t]
    base = tb_ref[t] * B
    rows = base + jax.lax.broadcasted_iota(jnp.int32, (B, 1), 0)
    mask = jnp.logical_and(rows >= rs, rows < re).astype(jnp.float32)

    xgb = xg_ref[...].astype(jnp.bfloat16)
    acc = jnp.zeros((B, D), jnp.float32)
    KC = 1024
    for kc in range(DFF // KC):
        gp = gp_ref[0, pl.ds(kc * KC, KC), :]
        up = up_ref[0, pl.ds(kc * KC, KC), :]
        h = jax.nn.silu(jax.lax.dot_general(
            xgb, gp, (((1,), (1,)), ((), ())),
            preferred_element_type=jnp.float32))
        h = h * jax.lax.dot_general(
            xgb, up, (((1,), (1,)), ((), ())),
            preferred_element_type=jnp.float32)
        dp = dp_ref[0, :, pl.ds(kc * KC, KC)]
        acc = acc + jax.lax.dot_general(
            h.astype(jnp.bfloat16), dp, (((1,), (1,)), ((), ())),
            preferred_element_type=jnp.float32)
    contrib = acc * mask

    prev = tb_ref[jnp.maximum(t - 1, 0)]
    new_blk = jnp.logical_or(t == 0, tb_ref[t] != prev)

    @pl.when(new_blk)
    def _init():
        y_ref[...] = contrib

    @pl.when(jnp.logical_not(new_blk))
    def _acc():
        y_ref[...] += contrib


def _route(gating_output):
    dest8, w8, cnt8 = pl.pallas_call(
        _route_kernel,
        out_shape=[
            jax.ShapeDtypeStruct((S, E), jnp.int32),
            jax.ShapeDtypeStruct((S, E), jnp.float32),
            jax.ShapeDtypeStruct((8, E), jnp.int32),
        ],
    )(gating_output)
    counts = cnt8[0]
    tb, te, trs, tre = pl.pallas_call(
        _tilemap_kernel,
        in_specs=[pl.BlockSpec(memory_space=pltpu.SMEM)],
        out_specs=[pl.BlockSpec(memory_space=pltpu.SMEM)] * 4,
        out_shape=[jax.ShapeDtypeStruct((NT,), jnp.int32)] * 4,
    )(counts)
    return dest8[:, 0], w8[:, 0], tb, te, trs, tre


def _gmm(xg, gate_b, up_b, down_b, tb, te, trs, tre):
    grid_spec = pltpu.PrefetchScalarGridSpec(
        num_scalar_prefetch=4,
        grid=(NT,),
        in_specs=[
            pl.BlockSpec((B, D), lambda t, tb, te, trs, tre: (tb[t], 0)),
            pl.BlockSpec((1, DFF, D), lambda t, tb, te, trs, tre: (te[t], 0, 0)),
            pl.BlockSpec((1, DFF, D), lambda t, tb, te, trs, tre: (te[t], 0, 0)),
            pl.BlockSpec((1, D, DFF), lambda t, tb, te, trs, tre: (te[t], 0, 0)),
        ],
        out_specs=pl.BlockSpec((B, D), lambda t, tb, te, trs, tre: (tb[t], 0)),
    )
    return pl.pallas_call(
        _gmm_kernel,
        grid_spec=grid_spec,
        out_shape=jax.ShapeDtypeStruct((S, D), jnp.float32),
        compiler_params=pltpu.CompilerParams(
            dimension_semantics=("arbitrary",),
        ),
    )(tb, te, trs, tre, xg, gate_b, up_b, down_b)


NW = 32
_SC_MESH = dict(core_axis_name="c", subcore_axis_name="s")
K2C = 64
K4C = 32


def _wid():
    return lax.axis_index("s") * 2 + lax.axis_index("c")


@functools.partial(
    pl.kernel,
    mesh=plsc.VectorSubcoreMesh(**_SC_MESH),
    out_type=jax.ShapeDtypeStruct((S, D), jnp.float32),
    scratch_types=[
        pltpu.VMEM((K2C,), jnp.int32),
        pltpu.VMEM((K2C, D), jnp.float32),
        pltpu.SemaphoreType.DMA,
    ],
)
def _sc_scatter(x_hbm, dest_hbm, xg_hbm, idx_v, rows_v, sem):
    wid = _wid()
    spw = S // NW
    sbase = wid * spw
    xbase = lax.rem(sbase, T)

    def chunk(c, _):
        off = c * K2C
        pltpu.sync_copy(dest_hbm.at[pl.ds(sbase + off, K2C)], idx_v)
        pltpu.sync_copy(x_hbm.at[pl.ds(xbase + off, K2C)], rows_v)
        pltpu.async_copy(rows_v, xg_hbm.at[idx_v], sem).wait()
        return 0

    lax.fori_loop(0, spw // K2C, chunk, 0)


@functools.partial(
    pl.kernel,
    mesh=plsc.VectorSubcoreMesh(**_SC_MESH),
    out_type=jax.ShapeDtypeStruct((T, D), jnp.float32),
    scratch_types=[
        pltpu.VMEM((K4C,), jnp.int32),
        pltpu.VMEM((K4C,), jnp.int32),
        pltpu.VMEM((K4C, D), jnp.float32),
        pltpu.VMEM((K4C, D), jnp.float32),
        pltpu.VMEM((64, 16), jnp.float32),
        pltpu.VMEM((64, 16), jnp.float32),
        pltpu.VMEM((K4C, D), jnp.float32),
        pltpu.SemaphoreType.DMA,
    ],
)
def _sc_combine(y_hbm, dest_hbm, w16_hbm, out_hbm,
                idx0_v, idx1_v, y0_v, y1_v, w0_v, w1_v, ob_v, sem):
    wid = _wid()
    tpw = T // NW
    tbase = wid * tpw
    NCOL = D // 16
    pltpu.sync_copy(w16_hbm.at[pl.ds(tbase, tpw)], w0_v)
    pltpu.sync_copy(w16_hbm.at[pl.ds(T + tbase, tpw)], w1_v)

    def chunk(c, _):
        t0 = tbase + c * K4C
        pltpu.sync_copy(dest_hbm.at[pl.ds(t0, K4C)], idx0_v)
        pltpu.sync_copy(dest_hbm.at[pl.ds(T + t0, K4C)], idx1_v)
        g0 = pltpu.async_copy(y_hbm.at[idx0_v], y0_v, sem)
        g1 = pltpu.async_copy(y_hbm.at[idx1_v], y1_v, sem)
        g0.wait()
        g1.wait()

        def row(j, _):
            jw = c * K4C + j
            w0 = w0_v[jw, :]
            w1 = w1_v[jw, :]

            def col(cc, _):
                sl = pl.ds(cc * 16, 16)
                ob_v[j, sl] = w0 * y0_v[j, sl] + w1 * y1_v[j, sl]
                return 0

            lax.fori_loop(0, NCOL, col, 0)
            return 0

        lax.fori_loop(0, K4C, row, 0)
        pltpu.sync_copy(ob_v, out_hbm.at[pl.ds(t0, K4C)])
        return 0

    lax.fori_loop(0, tpw // K4C, chunk, 0)


@functools.partial(jax.jit, static_argnums=())
def kernel(x, gating_output, gate_proj, up_proj, down_proj):
    dest, w_flat, tb, te, trs, tre = _route(gating_output)
    gate_b = gate_proj.astype(jnp.bfloat16)
    up_b = up_proj.astype(jnp.bfloat16)
    down_b = down_proj.astype(jnp.bfloat16)
    w16 = jnp.broadcast_to(w_flat[:, None], (S, 16))

    xg = _sc_scatter(x, dest)
    y = _gmm(xg, gate_b, up_b, down_b, tb, te, trs, tre)
    out = _sc_combine(y, dest, w16)
    return out
